# Initial kernel scaffold; baseline (speedup 1.0000x reference)
import jax
import jax.numpy as jnp
from jax.experimental import pallas as pl


def kernel(V, E, edges, params):
    raise NotImplementedError("write your pallas kernel here")


# speedup vs baseline: 5.2734x; 5.2734x over previous
import functools

import jax
import jax.numpy as jnp
from jax import lax
from jax.experimental import pallas as pl
from jax.experimental.pallas import tpu as pltpu
from jax.experimental.pallas import tpu_sc as plsc

F32 = jnp.float32
_NS = 16
_GK = 80


def _silu(x):
    return x * jax.nn.sigmoid(x)


def _ln(y, g, b):
    mu = jnp.mean(y, axis=-1, keepdims=True)
    yc = y - mu
    var = jnp.mean(yc * yc, axis=-1, keepdims=True)
    return yc * lax.rsqrt(var + 1e-5) * g + b


def _tc_project(V, w1s, w1r):
    n, node = V.shape
    bn = 2000
    hid = w1s.shape[1]

    def body(v_ref, ws_ref, wr_ref, ps_ref, pr_ref):
        v = v_ref[...]
        ps_ref[...] = jnp.dot(v, ws_ref[...], preferred_element_type=F32)
        pr_ref[...] = jnp.dot(v, wr_ref[...], preferred_element_type=F32)

    return pl.pallas_call(
        body,
        grid=(n // bn,),
        in_specs=[
            pl.BlockSpec((bn, node), lambda i: (i, 0)),
            pl.BlockSpec((node, hid), lambda i: (0, 0)),
            pl.BlockSpec((node, hid), lambda i: (0, 0)),
        ],
        out_specs=[
            pl.BlockSpec((bn, hid), lambda i: (i, 0)),
            pl.BlockSpec((bn, hid), lambda i: (i, 0)),
        ],
        out_shape=[
            jax.ShapeDtypeStruct((n, hid), F32),
            jax.ShapeDtypeStruct((n, hid), F32),
        ],
    )(V, w1s, w1r)


def _sc_gather(Ps, Pr, src, dst):
    n, d = Ps.shape
    ne = src.shape[0]
    per_tile = ne // _NS
    n_chunks = per_tile // _GK
    mesh = plsc.VectorSubcoreMesh(core_axis_name="c", subcore_axis_name="s")

    @functools.partial(
        pl.kernel,
        out_type=[
            jax.ShapeDtypeStruct((ne, d), F32),
            jax.ShapeDtypeStruct((ne, d), F32),
        ],
        mesh=mesh,
        scratch_types=[
            pltpu.VMEM((_GK,), jnp.int32),
            pltpu.VMEM((_GK, d), F32),
            pltpu.SemaphoreType.DMA,
        ],
        compiler_params=pltpu.CompilerParams(use_tc_tiling_on_sc=False),
    )
    def k(ps_hbm, pr_hbm, src_hbm, dst_hbm, gs_hbm, gr_hbm, idx_v, rows_v, sem):
        c = lax.axis_index("c")
        s = lax.axis_index("s")

        @pl.when(c == 0)
        def _():
            def body(i, carry):
                b = s * per_tile + i * _GK
                pltpu.sync_copy(src_hbm.at[pl.ds(b, _GK)], idx_v)
                pltpu.async_copy(ps_hbm.at[idx_v], rows_v, sem).wait()
                pltpu.sync_copy(rows_v, gs_hbm.at[pl.ds(b, _GK)])
                return carry

            lax.fori_loop(0, n_chunks, body, 0)

        @pl.when(c == 1)
        def _():
            def body(i, carry):
                b = s * per_tile + i * _GK
                pltpu.sync_copy(dst_hbm.at[pl.ds(b, _GK)], idx_v)
                pltpu.async_copy(pr_hbm.at[idx_v], rows_v, sem).wait()
                pltpu.sync_copy(rows_v, gr_hbm.at[pl.ds(b, _GK)])
                return carry

            lax.fori_loop(0, n_chunks, body, 0)

    return k(Ps, Pr, src, dst)


def _sc_scatter(ws, wr, src, dst, n):
    ne, d = ws.shape
    per_tile = ne // _NS
    n_chunks = per_tile // _GK
    rows_pt = n // _NS
    mesh = plsc.VectorSubcoreMesh(core_axis_name="c", subcore_axis_name="s")

    @functools.partial(
        pl.kernel,
        out_type=[
            jax.ShapeDtypeStruct((n, d), F32),
            jax.ShapeDtypeStruct((n, d), F32),
        ],
        mesh=mesh,
        scratch_types=[
            pltpu.VMEM((_GK,), jnp.int32),
            pltpu.VMEM((_GK, d), F32),
            pltpu.VMEM_SHARED((n, d), F32),
            pltpu.SemaphoreType.DMA,
        ],
        compiler_params=pltpu.CompilerParams(use_tc_tiling_on_sc=False),
    )
    def k(ws_hbm, wr_hbm, src_hbm, dst_hbm, z_hbm, outs_hbm, outr_hbm,
          idx_v, rows_v, acc_sh, sem):
        c = lax.axis_index("c")
        s = lax.axis_index("s")
        rs = pl.ds(s * rows_pt, rows_pt)
        pltpu.sync_copy(z_hbm.at[rs], acc_sh.at[rs])
        plsc.subcore_barrier()

        @pl.when(c == 0)
        def _():
            def body(i, carry):
                b = s * per_tile + i * _GK
                pltpu.sync_copy(src_hbm.at[pl.ds(b, _GK)], idx_v)
                pltpu.sync_copy(ws_hbm.at[pl.ds(b, _GK)], rows_v)
                pltpu.sync_copy(rows_v, acc_sh.at[idx_v], add=True)
                return carry

            lax.fori_loop(0, n_chunks, body, 0)

        @pl.when(c == 1)
        def _():
            def body(i, carry):
                b = s * per_tile + i * _GK
                pltpu.sync_copy(dst_hbm.at[pl.ds(b, _GK)], idx_v)
                pltpu.sync_copy(wr_hbm.at[pl.ds(b, _GK)], rows_v)
                pltpu.sync_copy(rows_v, acc_sh.at[idx_v], add=True)
                return carry

            lax.fori_loop(0, n_chunks, body, 0)

        plsc.subcore_barrier()

        @pl.when(c == 0)
        def _():
            pltpu.sync_copy(acc_sh.at[rs], outs_hbm.at[rs])

        @pl.when(c == 1)
        def _():
            pltpu.sync_copy(acc_sh.at[rs], outr_hbm.at[rs])

    return k(ws, wr, src, dst, jnp.zeros((n, d), F32))


def _tc_edge(Gs, Gr, E, w1e, b1e, w2e, b2e, ge, bee,
             w1m, b1m, w2ms, b2ms, gms, bms, w2mr, b2mr, gmr, bmr,
             w1a, b1a, w2as, b2as, w2ar, b2ar):
    ne, hid = Gs.shape
    edge = E.shape[1]
    be = 2000
    dw = 144

    def body(gs_ref, gr_ref, e_ref,
             w1e_r, b1e_r, w2e_r, b2e_r, ge_r, bee_r,
             w1m_r, b1m_r, w2ms_r, b2ms_r, gms_r, bms_r,
             w2mr_r, b2mr_r, gmr_r, bmr_r,
             w1a_r, b1a_r, w2as_r, b2as_r, w2ar_r, b2ar_r,
             emb_ref, ws_ref, wr_ref):
        h = gs_ref[...] + gr_ref[...] + b1e_r[...]
        h = h + jnp.dot(e_ref[...], w1e_r[...], preferred_element_type=F32)
        h = _silu(h)
        y = jnp.dot(h, w2e_r[...], preferred_element_type=F32) + b2e_r[...]
        emb = _ln(y, ge_r[...], bee_r[...])
        emb_ref[...] = emb

        hm = _silu(jnp.dot(emb, w1m_r[...], preferred_element_type=F32)
                   + b1m_r[...])
        ys = jnp.dot(hm[:, :hid], w2ms_r[...], preferred_element_type=F32) \
            + b2ms_r[...]
        yr = jnp.dot(hm[:, hid:], w2mr_r[...], preferred_element_type=F32) \
            + b2mr_r[...]
        ms = _ln(ys, gms_r[...], bms_r[...])
        mr = _ln(yr, gmr_r[...], bmr_r[...])

        ha = _silu(jnp.dot(emb, w1a_r[...], preferred_element_type=F32)
                   + b1a_r[...])
        ahid = w2as_r.shape[1]
        ls = jnp.sum(ha[:, :ahid] * w2as_r[...], axis=1, keepdims=True) \
            + b2as_r[...]
        lr = jnp.sum(ha[:, ahid:] * w2ar_r[...], axis=1, keepdims=True) \
            + b2ar_r[...]
        es = jnp.exp(jnp.clip(ls, -30.0, 30.0))
        er = jnp.exp(jnp.clip(lr, -30.0, 30.0))

        lane0 = (lax.broadcasted_iota(jnp.int32, (1, dw - hid), 1) == 0)
        lane0 = lane0.astype(F32)
        ws_ref[:, 0:hid] = ms * es
        ws_ref[:, hid:dw] = es * lane0
        wr_ref[:, 0:hid] = mr * er
        wr_ref[:, hid:dw] = er * lane0

    full = lambda shape: pl.BlockSpec(shape, lambda i: tuple(0 for _ in shape))
    return pl.pallas_call(
        body,
        grid=(ne // be,),
        in_specs=[
            pl.BlockSpec((be, hid), lambda i: (i, 0)),
            pl.BlockSpec((be, hid), lambda i: (i, 0)),
            pl.BlockSpec((be, edge), lambda i: (i, 0)),
        ] + [full(w.shape) for w in (
            w1e, b1e, w2e, b2e, ge, bee,
            w1m, b1m, w2ms, b2ms, gms, bms, w2mr, b2mr, gmr, bmr,
            w1a, b1a, w2as, b2as, w2ar, b2ar)],
        out_specs=[
            pl.BlockSpec((be, edge), lambda i: (i, 0)),
            pl.BlockSpec((be, dw), lambda i: (i, 0)),
            pl.BlockSpec((be, dw), lambda i: (i, 0)),
        ],
        out_shape=[
            jax.ShapeDtypeStruct((ne, edge), F32),
            jax.ShapeDtypeStruct((ne, dw), F32),
            jax.ShapeDtypeStruct((ne, dw), F32),
        ],
    )(Gs, Gr, E, w1e, b1e, w2e, b2e, ge, bee,
      w1m, b1m, w2ms, b2ms, gms, bms, w2mr, b2mr, gmr, bmr,
      w1a, b1a, w2as, b2as, w2ar, b2ar)


def _tc_node(V, accs, accr, w1v, w1s, w1r, b1, w2, b2, g, beta):
    n, node = V.shape
    hid = w1s.shape[0]
    dw = accs.shape[1]
    bn = 2000

    def body(v_ref, as_ref, ar_ref, w1v_r, w1s_r, w1r_r, b1_r, w2_r, b2_r,
             g_r, beta_r, out_ref):
        acs = as_ref[...]
        acr = ar_ref[...]
        aggs = acs[:, 0:hid] / (acs[:, hid:hid + 1] + 1e-30)
        aggr = acr[:, 0:hid] / (acr[:, hid:hid + 1] + 1e-30)
        h = jnp.dot(v_ref[...], w1v_r[...], preferred_element_type=F32)
        h = h + jnp.dot(aggs, w1s_r[...], preferred_element_type=F32)
        h = h + jnp.dot(aggr, w1r_r[...], preferred_element_type=F32)
        h = _silu(h + b1_r[...])
        y = jnp.dot(h, w2_r[...], preferred_element_type=F32) + b2_r[...]
        out_ref[...] = _ln(y, g_r[...], beta_r[...])

    full = lambda shape: pl.BlockSpec(shape, lambda i: tuple(0 for _ in shape))
    return pl.pallas_call(
        body,
        grid=(n // bn,),
        in_specs=[
            pl.BlockSpec((bn, node), lambda i: (i, 0)),
            pl.BlockSpec((bn, dw), lambda i: (i, 0)),
            pl.BlockSpec((bn, dw), lambda i: (i, 0)),
        ] + [full(w.shape) for w in (w1v, w1s, w1r, b1, w2, b2, g, beta)],
        out_specs=pl.BlockSpec((bn, node), lambda i: (i, 0)),
        out_shape=jax.ShapeDtypeStruct((n, node), F32),
    )(V, accs, accr, w1v, w1s, w1r, b1, w2, b2, g, beta)


def kernel(V, E, edges, params):
    V2 = V[0]
    E2 = E[0]
    src = edges[0, :, 0]
    dst = edges[0, :, 1]
    n, node = V2.shape

    pe = params["f_edge"]
    pms = params["f_msg_s"]
    pmr = params["f_msg_r"]
    pas = params["f_attn_s"]
    par = params["f_attn_r"]
    pn = params["f_node"]
    r1 = lambda a: a.reshape(1, -1)

    w1 = pe["w1"]
    Ps, Pr = _tc_project(V2, w1[:node], w1[node:2 * node])
    Gs, Gr = _sc_gather(Ps, Pr, src, dst)

    w1m = jnp.concatenate([pms["w1"], pmr["w1"]], axis=1)
    b1m = jnp.concatenate([pms["b1"], pmr["b1"]]).reshape(1, -1)
    w1a = jnp.concatenate([pas["w1"], par["w1"]], axis=1)
    b1a = jnp.concatenate([pas["b1"], par["b1"]]).reshape(1, -1)

    emb, ws, wr = _tc_edge(
        Gs, Gr, E2,
        w1[2 * node:], r1(pe["b1"]), pe["w2"], r1(pe["b2"]),
        r1(pe["g"]), r1(pe["beta"]),
        w1m, b1m, pms["w2"], r1(pms["b2"]), r1(pms["g"]), r1(pms["beta"]),
        pmr["w2"], r1(pmr["b2"]), r1(pmr["g"]), r1(pmr["beta"]),
        w1a, b1a, pas["w2"].reshape(1, -1), r1(pas["b2"]),
        par["w2"].reshape(1, -1), r1(par["b2"]))

    accs, accr = _sc_scatter(ws, wr, src, dst, n)

    wn = pn["w1"]
    hid = pe["w1"].shape[1]
    node_emb = _tc_node(
        V2, accs, accr,
        wn[:node], wn[node:node + hid], wn[node + hid:],
        r1(pn["b1"]), pn["w2"], r1(pn["b2"]), r1(pn["g"]), r1(pn["beta"]))

    return node_emb[None], emb[None]
